# baseline (speedup 1.0000x reference)
import functools

import jax
import jax.numpy as jnp
from jax import lax
from jax.experimental import pallas as pl
from jax.experimental.pallas import tpu as pltpu
from jax.experimental.pallas import tpu_sc as plsc

B, H, W = 8, 512, 512
K = 49
ID0 = 101
NC, NS, L = 2, 16, 16
NW = NC * NS
QP = NW // B
RPW = H // QP
CH = 16
NCH = RPW // CH
KW = K * W
KH = K * RPW

def _sc_hist_body(label_hbm, col_out, row_out, buf, colcnt, rowcnt, sem0, sem1):
    cid = lax.axis_index("c")
    sid = lax.axis_index("s")
    wid = sid * NC + cid
    b = wid // QP
    q = wid % QP
    row0 = q * RPW
    sems = (sem0, sem1)

    zz = jnp.zeros((L,), jnp.int32)
    ones = jnp.ones((L,), jnp.int32)
    riota = lax.iota(jnp.int32, L)

    @plsc.parallel_loop(0, KW // L, unroll=8)
    def _(i):
        colcnt[pl.ds(pl.multiple_of(i * L, 8), L)] = zz

    @plsc.parallel_loop(0, KH // L, unroll=8)
    def _(i):
        r = lax.shift_right_logical(i, 3)
        rowcnt[r, pl.ds(pl.multiple_of((i & 7) * L, 8), L)] = zz

    def dma(ci, slot):
        return pltpu.make_async_copy(
            label_hbm.at[b, pl.ds(row0 + ci * CH, CH)],
            buf.at[slot],
            sems[slot],
        )

    dma(0, 0).start()
    for ci in range(NCH):
        slot = ci & 1
        if ci + 1 < NCH:
            dma(ci + 1, (ci + 1) & 1).start()
        dma(ci, slot).wait()
        hloc = ci * CH + riota

        @plsc.parallel_loop(0, W, step=2, unroll=4, carry=(riota, riota))
        def _(i, c):
            wv, gcol = c
            r = lax.shift_right_logical(i, 5)
            woff = (i & 31) * L
            for u in (0, 1):
                labs = buf[slot, r, pl.ds(pl.multiple_of(woff + u * L, 8), L)]
                m = labs >= ID0
                plsc.addupdate_scatter(
                    colcnt, [(labs - ID0) * W + wv], ones, mask=m)
                labc = plsc.load_gather(buf.at[slot], [riota, gcol])
                mc = labc >= ID0
                plsc.addupdate_scatter(rowcnt, [labc - ID0, hloc], ones, mask=mc)
                wv = (wv + L) & (W - 1)
                gcol = (gcol + 1) & (W - 1)
            return (wv, gcol)

    pltpu.sync_copy(colcnt, col_out.at[b, q])
    pltpu.sync_copy(rowcnt, row_out.at[b, :, pl.ds(q * RPW, RPW)])


@functools.cache
def _sc_hist():
    mesh = plsc.VectorSubcoreMesh(
        core_axis_name="c", subcore_axis_name="s",
        num_cores=NC, num_subcores=NS)
    return pl.kernel(
        _sc_hist_body,
        out_type=(
            jax.ShapeDtypeStruct((B, QP, KW), jnp.int32),
            jax.ShapeDtypeStruct((B, K, H), jnp.int32),
        ),
        mesh=mesh,
        scratch_types=[
            pltpu.VMEM((2, CH, W), jnp.int32),
            pltpu.VMEM((KW,), jnp.int32),
            pltpu.VMEM((K, RPW), jnp.int32),
            pltpu.SemaphoreType.DMA,
            pltpu.SemaphoreType.DMA,
        ],
        compiler_params=pltpu.CompilerParams(needs_layout_passes=False),
    )


def _tc_body(ix_ref, iy_ref, pres_ref, logit_ref, out_ref):
    bidx = pl.program_id(0)
    wiota = lax.broadcasted_iota(jnp.int32, (1, W), 1)
    acc = jnp.zeros((1, W), jnp.float32)
    for k in range(K):
        row = logit_ref[0, pl.ds(iy_ref[bidx, k], 1), :]
        sel = (wiota == ix_ref[bidx, k]).astype(jnp.float32)
        acc += row * (sel * pres_ref[bidx, k])
    gsum = jnp.sum(acc)
    esum = jnp.sum(jnp.exp(logit_ref[0]))
    contrib = esum * (1.0 / (B * H * W)) - gsum * (1.0 / B)

    @pl.when(bidx == 0)
    def _():
        out_ref[...] = jnp.zeros_like(out_ref)

    out_ref[...] += contrib


_tc_finish = pl.pallas_call(
    _tc_body,
    out_shape=jax.ShapeDtypeStruct((1, 1), jnp.float32),
    grid=(B,),
    in_specs=[
        pl.BlockSpec(memory_space=pltpu.SMEM),
        pl.BlockSpec(memory_space=pltpu.SMEM),
        pl.BlockSpec(memory_space=pltpu.SMEM),
        pl.BlockSpec((1, H, W), lambda b: (b, 0, 0)),
    ],
    out_specs=pl.BlockSpec((1, 1), lambda b: (0, 0)),
)


def kernel(logit_score, label):
    colp, rowp = _sc_hist()(label)
    colc = colp.reshape(B, QP, K, W).sum(axis=1).astype(jnp.float32)
    rowc = rowp.astype(jnp.float32)
    com_x = jnp.arange(W, dtype=jnp.float32)[None, :]
    com_y = jnp.arange(H, dtype=jnp.float32)[None, :]
    ixs, iys, press = [], [], []
    for b in range(B):
        counts = colc[b].sum(axis=1)
        present = counts > 0
        safe_counts = jnp.where(present, counts, 1.0)
        cx = (com_x * colc[b] / safe_counts[:, None]).sum(axis=1)
        cy = (com_y * rowc[b] / safe_counts[:, None]).sum(axis=1)
        ixs.append(cx.astype(jnp.int32))
        iys.append(cy.astype(jnp.int32))
        press.append(present.astype(jnp.float32))
    ix = jnp.stack(ixs)
    iy = jnp.stack(iys)
    pres = jnp.stack(press)
    loss = _tc_finish(ix, iy, pres, logit_score)
    return loss[0, 0]


# speedup vs baseline: 1.0236x; 1.0236x over previous
import functools

import jax
import jax.numpy as jnp
from jax import lax
from jax.experimental import pallas as pl
from jax.experimental.pallas import tpu as pltpu
from jax.experimental.pallas import tpu_sc as plsc

B, H, W = 8, 512, 512
K = 49
ID0 = 101
NC, NS, L = 2, 16, 16
NW = NC * NS
QP = NW // B
RPW = H // QP
CH = 16
NCH = RPW // CH
KW = K * W
KH = K * RPW

def _sc_hist_body(label_hbm, col_out, row_out, buf, colcnt, rowcnt, sem0, sem1):
    cid = lax.axis_index("c")
    sid = lax.axis_index("s")
    wid = sid * NC + cid
    b = wid // QP
    q = wid % QP
    row0 = q * RPW
    sems = (sem0, sem1)

    zz = jnp.zeros((L,), jnp.int32)
    ones = jnp.ones((L,), jnp.int32)
    riota = lax.iota(jnp.int32, L)

    @plsc.parallel_loop(0, KW // L, unroll=8)
    def _(i):
        colcnt[pl.ds(pl.multiple_of(i * L, 8), L)] = zz

    @plsc.parallel_loop(0, KH // L, unroll=8)
    def _(i):
        r = lax.shift_right_logical(i, 3)
        rowcnt[r, pl.ds(pl.multiple_of((i & 7) * L, 8), L)] = zz

    def dma(ci, slot):
        return pltpu.make_async_copy(
            label_hbm.at[b, pl.ds(row0 + ci * CH, CH)],
            buf.at[slot],
            sems[slot],
        )

    dma(0, 0).start()
    for ci in range(NCH):
        slot = ci & 1
        if ci + 1 < NCH:
            dma(ci + 1, (ci + 1) & 1).start()
        dma(ci, slot).wait()
        hloc = ci * CH + riota

        @plsc.parallel_loop(0, W, unroll=8, carry=(riota, riota))
        def _(i, c):
            wv, gcol = c
            r = lax.shift_right_logical(i, 5)
            labs = buf[slot, r, pl.ds(pl.multiple_of((i & 31) * L, 8), L)]
            m = labs >= ID0
            plsc.addupdate_scatter(colcnt, [(labs - ID0) * W + wv], ones, mask=m)
            labc = plsc.load_gather(buf.at[slot], [riota, gcol])
            mc = labc >= ID0
            plsc.addupdate_scatter(rowcnt, [labc - ID0, hloc], ones, mask=mc)
            return ((wv + L) & (W - 1), (gcol + 1) & (W - 1))

    pltpu.sync_copy(colcnt, col_out.at[b, q])
    pltpu.sync_copy(rowcnt, row_out.at[b, :, pl.ds(q * RPW, RPW)])


@functools.cache
def _sc_hist():
    mesh = plsc.VectorSubcoreMesh(
        core_axis_name="c", subcore_axis_name="s",
        num_cores=NC, num_subcores=NS)
    return pl.kernel(
        _sc_hist_body,
        out_type=(
            jax.ShapeDtypeStruct((B, QP, KW), jnp.int32),
            jax.ShapeDtypeStruct((B, K, H), jnp.int32),
        ),
        mesh=mesh,
        scratch_types=[
            pltpu.VMEM((2, CH, W), jnp.int32),
            pltpu.VMEM((KW,), jnp.int32),
            pltpu.VMEM((K, RPW), jnp.int32),
            pltpu.SemaphoreType.DMA,
            pltpu.SemaphoreType.DMA,
        ],
        compiler_params=pltpu.CompilerParams(needs_layout_passes=False),
    )


def _tc_body(ix_ref, iy_ref, pres_ref, logit_ref, out_ref):
    bidx = pl.program_id(0)
    wiota = lax.broadcasted_iota(jnp.int32, (1, W), 1)
    acc = jnp.zeros((1, W), jnp.float32)
    for k in range(K):
        row = logit_ref[0, pl.ds(iy_ref[bidx, k], 1), :]
        sel = (wiota == ix_ref[bidx, k]).astype(jnp.float32)
        acc += row * (sel * pres_ref[bidx, k])
    gsum = jnp.sum(acc)
    esum = jnp.sum(jnp.exp(logit_ref[0]))
    contrib = esum * (1.0 / (B * H * W)) - gsum * (1.0 / B)

    @pl.when(bidx == 0)
    def _():
        out_ref[...] = jnp.zeros_like(out_ref)

    out_ref[...] += contrib


_tc_finish = pl.pallas_call(
    _tc_body,
    out_shape=jax.ShapeDtypeStruct((1, 1), jnp.float32),
    grid=(B,),
    in_specs=[
        pl.BlockSpec(memory_space=pltpu.SMEM),
        pl.BlockSpec(memory_space=pltpu.SMEM),
        pl.BlockSpec(memory_space=pltpu.SMEM),
        pl.BlockSpec((1, H, W), lambda b: (b, 0, 0)),
    ],
    out_specs=pl.BlockSpec((1, 1), lambda b: (0, 0)),
)


def kernel(logit_score, label):
    colp, rowp = _sc_hist()(label)
    colc = colp.reshape(B, QP, K, W).sum(axis=1).astype(jnp.float32)
    rowc = rowp.astype(jnp.float32)
    com_x = jnp.arange(W, dtype=jnp.float32)[None, :]
    com_y = jnp.arange(H, dtype=jnp.float32)[None, :]
    ixs, iys, press = [], [], []
    for b in range(B):
        counts = colc[b].sum(axis=1)
        present = counts > 0
        safe_counts = jnp.where(present, counts, 1.0)
        cx = (com_x * colc[b] / safe_counts[:, None]).sum(axis=1)
        cy = (com_y * rowc[b] / safe_counts[:, None]).sum(axis=1)
        ixs.append(cx.astype(jnp.int32))
        iys.append(cy.astype(jnp.int32))
        press.append(present.astype(jnp.float32))
    ix = jnp.stack(ixs)
    iy = jnp.stack(iys)
    pres = jnp.stack(press)
    loss = _tc_finish(ix, iy, pres, logit_score)
    return loss[0, 0]
